# baseline (speedup 1.0000x reference)
import jax
import jax.numpy as jnp
from jax.experimental import pallas as pl
from jax.experimental.pallas import tpu as pltpu

_BUDGET = 128.0
_SLOPE = 10.0


def _sample_body(mask_ref, sampler_ref, uniform_ref, probs_ref, nm_ref):
    mask_flat = mask_ref[...]
    so = jnp.broadcast_to(sampler_ref[...], mask_flat.shape)
    prob = jax.nn.softplus(_SLOPE * so) / _SLOPE
    denom = jnp.max((1.0 - mask_flat) * prob, axis=1, keepdims=True)
    prob = prob / denom
    masked = prob * (1.0 - mask_flat)
    sparsity = _BUDGET / mask_flat.shape[1]
    xbar = jnp.mean(masked, axis=1, keepdims=True)
    r = sparsity / xbar
    beta = (1.0 - sparsity) / (1.0 - xbar)
    le = (r <= 1.0).astype(masked.dtype)
    normed = le * masked * r + (1.0 - le) * (1.0 - (1.0 - masked) * beta)
    mprob = jnp.where(mask_flat == 0.0, normed, masked)
    binm = (mprob > uniform_ref[...]).astype(jnp.float32)
    probs_ref[...] = mprob
    nm_ref[...] = mask_flat + binm


def _mask_body(nm2_ref, x_ref, o_ref):
    o_ref[...] = x_ref[...] * nm2_ref[...]


def kernel(mask, kspace, sampler, uniform):
    B, M, H, W, C = kspace.shape
    mask_flat = mask.reshape(B, W)

    probs, nm = pl.pallas_call(
        _sample_body,
        out_shape=(
            jax.ShapeDtypeStruct((B, W), jnp.float32),
            jax.ShapeDtypeStruct((B, W), jnp.float32),
        ),
    )(mask_flat, sampler, uniform)

    nm2 = jnp.repeat(nm, C, axis=1).reshape(B, 1, W * C)

    MH = M * H
    HB = 512
    x = kspace.reshape(B, MH, W * C)
    out = pl.pallas_call(
        _mask_body,
        grid=(B, MH // HB),
        in_specs=[
            pl.BlockSpec((1, 1, W * C), lambda b, h: (b, 0, 0)),
            pl.BlockSpec((1, HB, W * C), lambda b, h: (b, h, 0)),
        ],
        out_specs=pl.BlockSpec((1, HB, W * C), lambda b, h: (b, h, 0)),
        out_shape=jax.ShapeDtypeStruct((B, MH, W * C), jnp.float32),
        compiler_params=pltpu.CompilerParams(
            dimension_semantics=("parallel", "parallel"),
        ),
    )(nm2, x)

    new_mask = nm.reshape(B, 1, 1, W, 1)
    final_prob_mask = probs.reshape(B, 1, 1, W, 1)
    return new_mask, out.reshape(B, M, H, W, C), final_prob_mask


# speedup vs baseline: 2.0106x; 2.0106x over previous
import jax
import jax.numpy as jnp
from jax.experimental import pallas as pl
from jax.experimental.pallas import tpu as pltpu

_BUDGET = 128.0
_SLOPE = 10.0


def _sample_body(mask_ref, sampler_ref, uniform_ref, probs_ref, nm_ref):
    mask_flat = mask_ref[...]
    so = jnp.broadcast_to(sampler_ref[...], mask_flat.shape)
    prob = jax.nn.softplus(_SLOPE * so) / _SLOPE
    denom = jnp.max((1.0 - mask_flat) * prob, axis=1, keepdims=True)
    prob = prob / denom
    masked = prob * (1.0 - mask_flat)
    sparsity = _BUDGET / mask_flat.shape[1]
    xbar = jnp.mean(masked, axis=1, keepdims=True)
    r = sparsity / xbar
    beta = (1.0 - sparsity) / (1.0 - xbar)
    le = (r <= 1.0).astype(masked.dtype)
    normed = le * masked * r + (1.0 - le) * (1.0 - (1.0 - masked) * beta)
    mprob = jnp.where(mask_flat == 0.0, normed, masked)
    binm = (mprob > uniform_ref[...]).astype(jnp.float32)
    probs_ref[...] = mprob
    nm_ref[...] = mask_flat + binm


def _mask_body(nm_ref, x_ref, o_ref):
    o_ref[...] = x_ref[...] * nm_ref[...]


def kernel(mask, kspace, sampler, uniform):
    B, M, H, W, C = kspace.shape
    mask_flat = mask.reshape(B, W)

    probs, nm = pl.pallas_call(
        _sample_body,
        out_shape=(
            jax.ShapeDtypeStruct((B, W), jnp.float32),
            jax.ShapeDtypeStruct((B, W), jnp.float32),
        ),
    )(mask_flat, sampler, uniform)

    x = kspace.transpose(0, 1, 2, 4, 3).reshape(B, M, H * C, W)
    nm3 = nm.reshape(B, 1, W)

    out = pl.pallas_call(
        _mask_body,
        grid=(B, M),
        in_specs=[
            pl.BlockSpec((1, 1, W), lambda b, m: (b, 0, 0)),
            pl.BlockSpec((1, 1, H * C, W), lambda b, m: (b, m, 0, 0)),
        ],
        out_specs=pl.BlockSpec((1, 1, H * C, W), lambda b, m: (b, m, 0, 0)),
        out_shape=jax.ShapeDtypeStruct((B, M, H * C, W), jnp.float32),
        compiler_params=pltpu.CompilerParams(
            dimension_semantics=("parallel", "parallel"),
        ),
    )(nm3, x)

    masked_kspace = out.reshape(B, M, H, C, W).transpose(0, 1, 2, 4, 3)
    new_mask = nm.reshape(B, 1, 1, W, 1)
    final_prob_mask = probs.reshape(B, 1, 1, W, 1)
    return new_mask, masked_kspace, final_prob_mask


# speedup vs baseline: 8.6514x; 4.3030x over previous
import jax
import jax.numpy as jnp
from jax.experimental import pallas as pl
from jax.experimental.pallas import tpu as pltpu

_BUDGET = 128.0
_SLOPE = 10.0


def _sample_body(mask_ref, sampler_ref, uniform_ref, probs_ref, nm_ref):
    mask_flat = mask_ref[...]
    so = jnp.broadcast_to(sampler_ref[...], mask_flat.shape)
    prob = jax.nn.softplus(_SLOPE * so) / _SLOPE
    denom = jnp.max((1.0 - mask_flat) * prob, axis=1, keepdims=True)
    prob = prob / denom
    masked = prob * (1.0 - mask_flat)
    sparsity = _BUDGET / mask_flat.shape[1]
    xbar = jnp.mean(masked, axis=1, keepdims=True)
    r = sparsity / xbar
    beta = (1.0 - sparsity) / (1.0 - xbar)
    le = (r <= 1.0).astype(masked.dtype)
    normed = le * masked * r + (1.0 - le) * (1.0 - (1.0 - masked) * beta)
    mprob = jnp.where(mask_flat == 0.0, normed, masked)
    binm = (mprob > uniform_ref[...]).astype(jnp.float32)
    probs_ref[...] = mprob
    nm_ref[...] = mask_flat + binm


def _mask_body(nm_ref, x_ref, o_ref):
    o_ref[0, 0] = x_ref[0, 0] * nm_ref[...]


def kernel(mask, kspace, sampler, uniform):
    B, M, H, W, C = kspace.shape
    mask_flat = mask.reshape(B, W)

    probs, nm = pl.pallas_call(
        _sample_body,
        out_shape=(
            jax.ShapeDtypeStruct((B, W), jnp.float32),
            jax.ShapeDtypeStruct((B, W), jnp.float32),
        ),
    )(mask_flat, sampler, uniform)

    x = kspace.transpose(0, 1, 2, 4, 3)
    nm3 = nm.reshape(B, 1, W)

    out = pl.pallas_call(
        _mask_body,
        grid=(B, M),
        in_specs=[
            pl.BlockSpec((1, 1, W), lambda b, m: (b, 0, 0)),
            pl.BlockSpec((1, 1, H, C, W), lambda b, m: (b, m, 0, 0, 0)),
        ],
        out_specs=pl.BlockSpec((1, 1, H, C, W), lambda b, m: (b, m, 0, 0, 0)),
        out_shape=jax.ShapeDtypeStruct((B, M, H, C, W), jnp.float32),
        compiler_params=pltpu.CompilerParams(
            dimension_semantics=("parallel", "parallel"),
        ),
    )(nm3, x)

    masked_kspace = out.transpose(0, 1, 2, 4, 3)
    new_mask = nm.reshape(B, 1, 1, W, 1)
    final_prob_mask = probs.reshape(B, 1, 1, W, 1)
    return new_mask, masked_kspace, final_prob_mask


# speedup vs baseline: 9.4013x; 1.0867x over previous
import jax
import jax.numpy as jnp
from jax.experimental import pallas as pl
from jax.experimental.pallas import tpu as pltpu

_BUDGET = 128.0
_SLOPE = 10.0


def _sample_body(mask_ref, sampler_ref, uniform_ref, probs_ref, nm_ref):
    mask_flat = mask_ref[...]
    so = jnp.broadcast_to(sampler_ref[...], mask_flat.shape)
    prob = jax.nn.softplus(_SLOPE * so) / _SLOPE
    denom = jnp.max((1.0 - mask_flat) * prob, axis=1, keepdims=True)
    prob = prob / denom
    masked = prob * (1.0 - mask_flat)
    sparsity = _BUDGET / mask_flat.shape[1]
    xbar = jnp.mean(masked, axis=1, keepdims=True)
    r = sparsity / xbar
    beta = (1.0 - sparsity) / (1.0 - xbar)
    le = (r <= 1.0).astype(masked.dtype)
    normed = le * masked * r + (1.0 - le) * (1.0 - (1.0 - masked) * beta)
    mprob = jnp.where(mask_flat == 0.0, normed, masked)
    binm = (mprob > uniform_ref[...]).astype(jnp.float32)
    probs_ref[...] = mprob
    nm_ref[...] = mask_flat + binm


def _mask_body(nm_ref, x_ref, o_ref):
    o_ref[0] = x_ref[0] * nm_ref[...]


def kernel(mask, kspace, sampler, uniform):
    B, M, H, W, C = kspace.shape
    mask_flat = mask.reshape(B, W)

    probs, nm = pl.pallas_call(
        _sample_body,
        out_shape=(
            jax.ShapeDtypeStruct((B, W), jnp.float32),
            jax.ShapeDtypeStruct((B, W), jnp.float32),
        ),
    )(mask_flat, sampler, uniform)

    x = kspace.transpose(0, 1, 2, 4, 3)
    nm3 = nm.reshape(B, 1, W)

    MB = 2
    out = pl.pallas_call(
        _mask_body,
        grid=(B, M // MB),
        in_specs=[
            pl.BlockSpec((1, 1, W), lambda b, m: (b, 0, 0)),
            pl.BlockSpec((1, MB, H, C, W), lambda b, m: (b, m, 0, 0, 0)),
        ],
        out_specs=pl.BlockSpec((1, MB, H, C, W), lambda b, m: (b, m, 0, 0, 0)),
        out_shape=jax.ShapeDtypeStruct((B, M, H, C, W), jnp.float32),
        compiler_params=pltpu.CompilerParams(
            dimension_semantics=("parallel", "parallel"),
        ),
    )(nm3, x)

    masked_kspace = out.transpose(0, 1, 2, 4, 3)
    new_mask = nm.reshape(B, 1, 1, W, 1)
    final_prob_mask = probs.reshape(B, 1, 1, W, 1)
    return new_mask, masked_kspace, final_prob_mask


# speedup vs baseline: 9.5003x; 1.0105x over previous
import jax
import jax.numpy as jnp
from jax.experimental import pallas as pl
from jax.experimental.pallas import tpu as pltpu

_BUDGET = 128.0
_SLOPE = 10.0


def _sample_body(mask_ref, sampler_ref, uniform_ref, probs_ref, nm_ref):
    mask_flat = mask_ref[...]
    so = jnp.broadcast_to(sampler_ref[...], mask_flat.shape)
    prob = jax.nn.softplus(_SLOPE * so) / _SLOPE
    denom = jnp.max((1.0 - mask_flat) * prob, axis=1, keepdims=True)
    prob = prob / denom
    masked = prob * (1.0 - mask_flat)
    sparsity = _BUDGET / mask_flat.shape[1]
    xbar = jnp.mean(masked, axis=1, keepdims=True)
    r = sparsity / xbar
    beta = (1.0 - sparsity) / (1.0 - xbar)
    le = (r <= 1.0).astype(masked.dtype)
    normed = le * masked * r + (1.0 - le) * (1.0 - (1.0 - masked) * beta)
    mprob = jnp.where(mask_flat == 0.0, normed, masked)
    binm = (mprob > uniform_ref[...]).astype(jnp.float32)
    probs_ref[...] = mprob
    nm_ref[...] = mask_flat + binm


def _mask_body(nm_ref, x_ref, o_ref):
    o_ref[0] = x_ref[0] * nm_ref[...]


def kernel(mask, kspace, sampler, uniform):
    B, M, H, W, C = kspace.shape
    mask_flat = mask.reshape(B, W)

    probs, nm = pl.pallas_call(
        _sample_body,
        out_shape=(
            jax.ShapeDtypeStruct((B, W), jnp.float32),
            jax.ShapeDtypeStruct((B, W), jnp.float32),
        ),
    )(mask_flat, sampler, uniform)

    x = kspace.transpose(0, 1, 2, 4, 3)
    nm3 = nm.reshape(B, 1, W)

    MB = 4
    out = pl.pallas_call(
        _mask_body,
        grid=(B, M // MB),
        in_specs=[
            pl.BlockSpec((1, 1, W), lambda b, m: (b, 0, 0)),
            pl.BlockSpec((1, MB, H, C, W), lambda b, m: (b, m, 0, 0, 0)),
        ],
        out_specs=pl.BlockSpec((1, MB, H, C, W), lambda b, m: (b, m, 0, 0, 0)),
        out_shape=jax.ShapeDtypeStruct((B, M, H, C, W), jnp.float32),
        compiler_params=pltpu.CompilerParams(
            dimension_semantics=("parallel", "parallel"),
        ),
    )(nm3, x)

    masked_kspace = out.transpose(0, 1, 2, 4, 3)
    new_mask = nm.reshape(B, 1, 1, W, 1)
    final_prob_mask = probs.reshape(B, 1, 1, W, 1)
    return new_mask, masked_kspace, final_prob_mask


# speedup vs baseline: 9.6800x; 1.0189x over previous
import jax
import jax.numpy as jnp
from jax.experimental import pallas as pl
from jax.experimental.pallas import tpu as pltpu

_BUDGET = 128.0
_SLOPE = 10.0


def _body(mask_ref, sampler_ref, uniform_ref, x_ref, probs_ref, nm_out_ref,
          o_ref, nm_s):
    b = pl.program_id(0)
    m = pl.program_id(1)

    @pl.when((b == 0) & (m == 0))
    def _sample():
        mask_flat = mask_ref[...]
        so = jnp.broadcast_to(sampler_ref[...], mask_flat.shape)
        prob = jax.nn.softplus(_SLOPE * so) / _SLOPE
        denom = jnp.max((1.0 - mask_flat) * prob, axis=1, keepdims=True)
        prob = prob / denom
        masked = prob * (1.0 - mask_flat)
        sparsity = _BUDGET / mask_flat.shape[1]
        xbar = jnp.mean(masked, axis=1, keepdims=True)
        r = sparsity / xbar
        beta = (1.0 - sparsity) / (1.0 - xbar)
        le = (r <= 1.0).astype(masked.dtype)
        normed = le * masked * r + (1.0 - le) * (1.0 - (1.0 - masked) * beta)
        mprob = jnp.where(mask_flat == 0.0, normed, masked)
        binm = (mprob > uniform_ref[...]).astype(jnp.float32)
        nm = mask_flat + binm
        probs_ref[...] = mprob
        nm_out_ref[...] = nm
        nm_s[...] = nm

    row = nm_s[pl.ds(b, 1), :]
    o_ref[0] = x_ref[0] * row


def kernel(mask, kspace, sampler, uniform):
    B, M, H, W, C = kspace.shape
    mask_flat = mask.reshape(B, W)

    x = kspace.transpose(0, 1, 2, 4, 3)

    MB = 4
    probs, nm, out = pl.pallas_call(
        _body,
        grid=(B, M // MB),
        in_specs=[
            pl.BlockSpec((B, W), lambda b, m: (0, 0)),
            pl.BlockSpec((1, W), lambda b, m: (0, 0)),
            pl.BlockSpec((B, W), lambda b, m: (0, 0)),
            pl.BlockSpec((1, MB, H, C, W), lambda b, m: (b, m, 0, 0, 0)),
        ],
        out_specs=(
            pl.BlockSpec((B, W), lambda b, m: (0, 0)),
            pl.BlockSpec((B, W), lambda b, m: (0, 0)),
            pl.BlockSpec((1, MB, H, C, W), lambda b, m: (b, m, 0, 0, 0)),
        ),
        out_shape=(
            jax.ShapeDtypeStruct((B, W), jnp.float32),
            jax.ShapeDtypeStruct((B, W), jnp.float32),
            jax.ShapeDtypeStruct((B, M, H, C, W), jnp.float32),
        ),
        scratch_shapes=[pltpu.VMEM((B, W), jnp.float32)],
        compiler_params=pltpu.CompilerParams(
            dimension_semantics=("arbitrary", "arbitrary"),
        ),
    )(mask_flat, sampler, uniform, x)

    masked_kspace = out.transpose(0, 1, 2, 4, 3)
    new_mask = nm.reshape(B, 1, 1, W, 1)
    final_prob_mask = probs.reshape(B, 1, 1, W, 1)
    return new_mask, masked_kspace, final_prob_mask
